# baseline (speedup 1.0000x reference)
import functools
import math

import jax
import jax.numpy as jnp
import numpy as np
from jax import lax
from jax.experimental import pallas as pl
from jax.experimental.pallas import tpu as pltpu
from jax.experimental.pallas import tpu_sc as plsc

D_MODEL = 768
MASK_TOKEN_ID = 4

NUM_CORES = 2
NUM_SUBCORES = 16
NUM_WORKERS = NUM_CORES * NUM_SUBCORES
CHUNK = 32
NB = 4


def _pe_table(seq_len: int, d_model: int) -> np.ndarray:
    pos = np.arange(seq_len, dtype=np.float32)[:, None]
    div = np.exp(
        np.arange(0, d_model, 2, dtype=np.float32) * (-(math.log(10000.0) / d_model))
    )
    pe = np.zeros((seq_len, d_model), dtype=np.float32)
    pe[:, 0::2] = np.sin(pos * div)
    pe[:, 1::2] = np.cos(pos * div)
    return pe


def _pe_prepped(seq_len: int, d_model: int) -> np.ndarray:
    import ml_dtypes

    pe = _pe_table(seq_len, d_model)
    x = pe.reshape(seq_len, d_model // 32, 2, 16)
    a = x[:, :, 0, :].astype(ml_dtypes.bfloat16).view(np.uint16).astype(np.uint32)
    b = x[:, :, 1, :].astype(ml_dtypes.bfloat16).view(np.uint16).astype(np.uint32)
    words = a | (b << 16)
    return words.view(np.float32).reshape(seq_len, d_model // 2)


@functools.cache
def _build_sc_gather(batch: int, seq_len: int, d: int, ext_row: int):
    ppw = seq_len // NUM_WORKERS
    halves = ppw // CHUNK
    n_sub = batch * halves
    mesh = plsc.VectorSubcoreMesh(core_axis_name="c", subcore_axis_name="s")

    @functools.partial(
        pl.kernel,
        out_type=jax.ShapeDtypeStruct((batch * seq_len, d), jnp.float32),
        mesh=mesh,
        scratch_types=[
            pltpu.VMEM((batch, ppw), jnp.int32),
            pltpu.VMEM((ppw, d // 2), jnp.float32),
            pltpu.VMEM((NB, CHUNK, d), jnp.float32),
        ]
        + [pltpu.SemaphoreType.DMA] * (2 + 2 * NB),
    )
    def sc_gather(table_hbm, ids_hbm, pe_hbm, out_hbm, ids_v, pe_v, gbuf, *sems):
        pesem = sems[0]
        isem = sems[1]
        gsems = sems[2 : 2 + NB]
        osems = sems[2 + NB : 2 + 2 * NB]
        wid = lax.axis_index("s") * NUM_CORES + lax.axis_index("c")
        pos0 = pl.multiple_of(wid * ppw, ppw)

        pe_done = pltpu.async_copy(pe_hbm.at[pl.ds(pos0, ppw)], pe_v, pesem)
        ids_descs = [
            pltpu.async_copy(ids_hbm.at[b].at[pl.ds(pos0, ppw)], ids_v.at[b], isem)
            for b in range(batch)
        ]
        for dsc in ids_descs:
            dsc.wait()

        def start_gather(c):
            b, h = divmod(c, halves)
            return pltpu.async_copy(
                table_hbm.at[ids_v.at[b].at[pl.ds(h * CHUNK, CHUNK)]],
                gbuf.at[c % NB],
                gsems[c % NB],
            )

        def start_out(c):
            b, h = divmod(c, halves)
            return pltpu.async_copy(
                gbuf.at[c % NB],
                out_hbm.at[pl.ds(b * seq_len + pos0 + h * CHUNK, CHUNK)],
                osems[c % NB],
            )

        g_desc = [None] * n_sub
        o_desc = [None] * n_sub
        for c in range(min(NB - 1, n_sub)):
            g_desc[c] = start_gather(c)
        pe_done.wait()

        for c in range(n_sub):
            g_desc[c].wait()
            if c + NB - 1 < n_sub:
                if c - 1 >= 0:
                    o_desc[c - 1].wait()
                g_desc[c + NB - 1] = start_gather(c + NB - 1)
            h = c % halves
            dst = gbuf.at[c % NB]

            def row_add(r, carry, h=h, dst=dst):
                pr = pe_v.at[h * CHUNK + r]
                dr = dst.at[r]
                for i in range(d // 32):
                    w = lax.bitcast_convert_type(
                        pr[pl.ds(i * 16, 16)], jnp.int32
                    )
                    a = lax.bitcast_convert_type(w << 16, jnp.float32)
                    bb = lax.bitcast_convert_type(
                        w & jnp.int32(-65536), jnp.float32
                    )
                    plsc.addupdate(dr.at[pl.ds(i * 32, 16)], a)
                    plsc.addupdate(dr.at[pl.ds(i * 32 + 16, 16)], bb)
                return carry

            lax.fori_loop(0, CHUNK, row_add, 0)
            o_desc[c] = start_out(c)
        for c in range(max(0, n_sub - NB), n_sub):
            o_desc[c].wait()

    return sc_gather


def kernel(input_ids, table, mask_token):
    b, l = input_ids.shape
    v, d = table.shape
    tbl = table.at[MASK_TOKEN_ID].set(mask_token)
    ids = input_ids.astype(jnp.int32)
    pe = jnp.asarray(_pe_prepped(l, d))
    out = _build_sc_gather(b, l, d, v)(tbl, ids, pe)
    return out.reshape(b, l, d)


# speedup vs baseline: 1.0659x; 1.0659x over previous
import functools
import math

import jax
import jax.numpy as jnp
import numpy as np
from jax import lax
from jax.experimental import pallas as pl
from jax.experimental.pallas import tpu as pltpu
from jax.experimental.pallas import tpu_sc as plsc

D_MODEL = 768
MASK_TOKEN_ID = 4

NUM_CORES = 2
NUM_SUBCORES = 16
NUM_WORKERS = NUM_CORES * NUM_SUBCORES
CHUNK = 32
NB = 4


def _pe_table(seq_len: int, d_model: int) -> np.ndarray:
    pos = np.arange(seq_len, dtype=np.float32)[:, None]
    div = np.exp(
        np.arange(0, d_model, 2, dtype=np.float32) * (-(math.log(10000.0) / d_model))
    )
    pe = np.zeros((seq_len, d_model), dtype=np.float32)
    pe[:, 0::2] = np.sin(pos * div)
    pe[:, 1::2] = np.cos(pos * div)
    return pe


def _pe_prepped(seq_len: int, d_model: int) -> np.ndarray:
    import ml_dtypes

    pe = _pe_table(seq_len, d_model)
    x = pe.reshape(seq_len, d_model // 32, 2, 16)
    a = x[:, :, 0, :].astype(ml_dtypes.bfloat16).view(np.uint16).astype(np.uint32)
    b = x[:, :, 1, :].astype(ml_dtypes.bfloat16).view(np.uint16).astype(np.uint32)
    words = a | (b << 16)
    return words.view(np.float32).reshape(seq_len, d_model // 2)


@functools.cache
def _build_sc_gather(batch: int, seq_len: int, d: int, ext_row: int):
    ppw = seq_len // NUM_WORKERS
    halves = ppw // CHUNK
    n_sub = batch * halves
    mesh = plsc.VectorSubcoreMesh(core_axis_name="c", subcore_axis_name="s")

    @functools.partial(
        pl.kernel,
        out_type=jax.ShapeDtypeStruct((batch * seq_len, d), jnp.float32),
        mesh=mesh,
        scratch_types=[
            pltpu.VMEM((batch, ppw), jnp.int32),
            pltpu.VMEM((ppw, d // 2), jnp.float32),
            pltpu.VMEM((NB, CHUNK, d), jnp.float32),
        ]
        + [pltpu.SemaphoreType.DMA] * (2 + 2 * NB),
    )
    def sc_gather(table_hbm, ids_hbm, pe_hbm, out_hbm, ids_v, pe_v, gbuf, *sems):
        pesem = sems[0]
        isem = sems[1]
        gsems = sems[2 : 2 + NB]
        osems = sems[2 + NB : 2 + 2 * NB]
        wid = lax.axis_index("s") * NUM_CORES + lax.axis_index("c")
        pos0 = pl.multiple_of(wid * ppw, ppw)

        pe_done = pltpu.async_copy(pe_hbm.at[pl.ds(pos0, ppw)], pe_v, pesem)
        ids_descs = [
            pltpu.async_copy(ids_hbm.at[b].at[pl.ds(pos0, ppw)], ids_v.at[b], isem)
            for b in range(batch)
        ]
        for dsc in ids_descs:
            dsc.wait()

        def start_gather(c):
            b, h = divmod(c, halves)
            return pltpu.async_copy(
                table_hbm.at[ids_v.at[b].at[pl.ds(h * CHUNK, CHUNK)]],
                gbuf.at[c % NB],
                gsems[c % NB],
            )

        def start_out(c):
            b, h = divmod(c, halves)
            return pltpu.async_copy(
                gbuf.at[c % NB],
                out_hbm.at[pl.ds(b * seq_len + pos0 + h * CHUNK, CHUNK)],
                osems[c % NB],
            )

        g_desc = [None] * n_sub
        o_desc = [None] * n_sub
        for c in range(min(NB - 1, n_sub)):
            g_desc[c] = start_gather(c)
        pe_done.wait()

        for c in range(n_sub):
            g_desc[c].wait()
            h = c % halves
            dst = gbuf.at[c % NB]

            def row_add(r, carry, h=h, dst=dst):
                pr = pe_v.at[h * CHUNK + r]
                dr = dst.at[r]
                for i in range(d // 32):
                    w = lax.bitcast_convert_type(
                        pr[pl.ds(i * 16, 16)], jnp.int32
                    )
                    a = lax.bitcast_convert_type(w << 16, jnp.float32)
                    bb = lax.bitcast_convert_type(
                        w & jnp.int32(-65536), jnp.float32
                    )
                    plsc.addupdate(dr.at[pl.ds(i * 32, 16)], a)
                    plsc.addupdate(dr.at[pl.ds(i * 32 + 16, 16)], bb)
                return carry

            lax.fori_loop(0, CHUNK, row_add, 0)
            o_desc[c] = start_out(c)
            if c + NB - 1 < n_sub:
                if c - 1 >= 0:
                    o_desc[c - 1].wait()
                g_desc[c + NB - 1] = start_gather(c + NB - 1)
        for c in range(max(0, n_sub - NB), n_sub):
            o_desc[c].wait()

    return sc_gather


def kernel(input_ids, table, mask_token):
    b, l = input_ids.shape
    v, d = table.shape
    tbl = table.at[MASK_TOKEN_ID].set(mask_token)
    ids = input_ids.astype(jnp.int32)
    pe = jnp.asarray(_pe_prepped(l, d))
    out = _build_sc_gather(b, l, d, v)(tbl, ids, pe)
    return out.reshape(b, l, d)
